# Initial kernel scaffold; baseline (speedup 1.0000x reference)
import jax
import jax.numpy as jnp
from jax.experimental import pallas as pl


def kernel(x):
    raise NotImplementedError("write your pallas kernel here")


# speedup vs baseline: 28.6046x; 28.6046x over previous
import functools

import jax
import jax.numpy as jnp
from jax.experimental import pallas as pl
from jax.experimental.pallas import tpu as pltpu

K_CLUSTERS = 512
N_TOKENS = 16384
D_CODE = 64
D_AUG = 128
MAX_ITERS = 100
BLK = 2048
NBLK = N_TOKENS // BLK
RTOL = 1e-4
ATOL = 1e-8


def _kmeans_loss_kernel(xa_ref, c0_ref, out_ref, ca_ref, ids_ref):
    iota_k = jax.lax.broadcasted_iota(jnp.int32, (BLK, K_CLUSTERS), 1)
    ca_ref[...] = c0_ref[...]

    def body(carry):
        it, _ = carry
        ca = ca_ref[...]
        c = ca[:, :D_CODE]
        stats = jnp.zeros((K_CLUSTERS, D_AUG), jnp.float32)
        for blk in range(NBLK):
            xa = xa_ref[blk * BLK:(blk + 1) * BLK, :]
            score = jax.lax.dot_general(xa, ca, (((1,), (1,)), ((), ())),
                                        preferred_element_type=jnp.float32)
            maxval = jnp.max(score, axis=1, keepdims=True)
            ids = jnp.min(jnp.where(score == maxval, iota_k, K_CLUSTERS),
                          axis=1, keepdims=True)
            ids_ref[blk * BLK:(blk + 1) * BLK, :] = ids
            onehot = (ids == iota_k).astype(jnp.float32)
            stats = stats + jax.lax.dot_general(
                onehot, xa, (((0,), (0,)), ((), ())),
                preferred_element_type=jnp.float32)
        sums = stats[:, :D_CODE]
        counts = stats[:, D_CODE:D_CODE + 1]
        new_c = sums / jnp.maximum(counts, 1.0)
        keep = (counts <= 0.0).astype(jnp.float32)
        new_c = new_c + (c - new_c) * keep
        viol = jnp.max(jnp.abs(c - new_c) - (ATOL + RTOL * jnp.abs(new_c)))
        converged = (viol <= 0.0).astype(jnp.int32)
        cf = converged.astype(jnp.float32)
        upd = new_c + (c - new_c) * cf
        ca_ref[:, :D_CODE] = upd
        ca_ref[:, D_CODE:D_CODE + 1] = -0.5 * jnp.sum(upd * upd, axis=1,
                                                      keepdims=True)
        return it + 1, converged

    jax.lax.while_loop(
        lambda carry: jnp.logical_and(carry[0] < MAX_ITERS, carry[1] == 0),
        body, (jnp.int32(0), jnp.int32(0)))

    c = ca_ref[:, :D_CODE]
    total = jnp.float32(0.0)
    for blk in range(NBLK):
        xb = xa_ref[blk * BLK:(blk + 1) * BLK, :D_CODE]
        m = jax.lax.dot_general(xb, c, (((1,), (1,)), ((), ())),
                                preferred_element_type=jnp.float32)
        rowmax = jnp.max(m, axis=1, keepdims=True)
        lse = jnp.log(jnp.sum(jnp.exp(m - rowmax), axis=1,
                              keepdims=True)) + rowmax
        onehot = (ids_ref[blk * BLK:(blk + 1) * BLK, :] ==
                  iota_k).astype(jnp.float32)
        label_logit = jnp.sum(m * onehot, axis=1, keepdims=True)
        total += jnp.sum(lse - label_logit)
    out_ref[0, 0] = total / jnp.float32(N_TOKENS)


@functools.partial(jax.jit, static_argnames=("interpret",))
def kernel(x, interpret=False):
    perm = jax.random.permutation(jax.random.key(42), N_TOKENS)
    c0 = x[perm[:K_CLUSTERS]]
    ones = jnp.ones((N_TOKENS, 1), jnp.float32)
    zpad_x = jnp.zeros((N_TOKENS, D_AUG - D_CODE - 1), jnp.float32)
    xa = jnp.concatenate([x, ones, zpad_x], axis=1)
    b2 = -0.5 * jnp.sum(c0 * c0, axis=1, keepdims=True)
    zpad_c = jnp.zeros((K_CLUSTERS, D_AUG - D_CODE - 1), jnp.float32)
    c0a = jnp.concatenate([c0, b2, zpad_c], axis=1)
    loss = pl.pallas_call(
        _kmeans_loss_kernel,
        out_shape=jax.ShapeDtypeStruct((1, 1), jnp.float32),
        in_specs=[pl.BlockSpec(memory_space=pltpu.VMEM),
                  pl.BlockSpec(memory_space=pltpu.VMEM)],
        out_specs=pl.BlockSpec(memory_space=pltpu.SMEM),
        scratch_shapes=[
            pltpu.VMEM((K_CLUSTERS, D_AUG), jnp.float32),
            pltpu.VMEM((N_TOKENS, 1), jnp.int32),
        ],
        interpret=interpret,
    )(xa, c0a)
    return jnp.reshape(loss, ())
